# baseline (speedup 1.0000x reference)
import functools

import jax
import jax.numpy as jnp
from jax.experimental import pallas as pl
from jax.experimental.pallas import tpu as pltpu
from jax.sharding import Mesh, PartitionSpec as P
from jax.experimental.shard_map import shard_map

TOKENS = 16384
HIDDEN = 2048
EXPERTS = 64
BLOCK_T = 1024


def _router_body(x_ref, w_ref, b_ref, o_ref):
    x = x_ref[...].astype(jnp.bfloat16)
    w = w_ref[...].astype(jnp.bfloat16)
    logits = jax.lax.dot_general(
        x, w,
        dimension_numbers=(((1,), (1,)), ((), ())),
        preferred_element_type=jnp.float32,
    )
    logits = logits + b_ref[...]
    o_ref[...] = jnp.maximum(logits, 0.0)


def _router_shard(x, W, b2):
    tokens = x.shape[0]
    grid = (tokens // BLOCK_T,)
    return pl.pallas_call(
        _router_body,
        grid=grid,
        in_specs=[
            pl.BlockSpec((BLOCK_T, HIDDEN), lambda i: (i, 0)),
            pl.BlockSpec((EXPERTS, HIDDEN), lambda i: (0, 0)),
            pl.BlockSpec((1, EXPERTS), lambda i: (0, 0)),
        ],
        out_specs=pl.BlockSpec((BLOCK_T, EXPERTS), lambda i: (i, 0)),
        out_shape=jax.ShapeDtypeStruct((tokens, EXPERTS), jnp.float32),
        compiler_params=pltpu.CompilerParams(
            dimension_semantics=("parallel",),
        ),
    )(x, W, b2)


@jax.jit
def kernel(x, W, b):
    b2 = b.reshape(1, EXPERTS)
    n_dev = len(jax.devices())
    n_dp = 2 if (n_dev >= 2 and TOKENS % (2 * BLOCK_T) == 0) else 1
    if n_dp == 1:
        return _router_shard(x, W, b2)
    mesh = Mesh(jax.devices()[:n_dp], ("dp",))
    f = shard_map(
        _router_shard,
        mesh=mesh,
        in_specs=(P("dp", None), P(None, None), P(None, None)),
        out_specs=P("dp", None),
        check_rep=False,
    )
    return f(x, W, b2)


# speedup vs baseline: 9.5813x; 9.5813x over previous
import jax
import jax.numpy as jnp
from jax.experimental import pallas as pl
from jax.experimental.pallas import tpu as pltpu

TOKENS = 16384
HIDDEN = 2048
EXPERTS = 64
BLOCK_T = 1024
NBLOCKS = TOKENS // BLOCK_T
NBUF = 3


def _router_body(x_hbm, w_ref, b_ref, o_ref, xbuf, sems):
    w = w_ref[...].astype(jnp.bfloat16)
    bias = b_ref[...]

    def copy_in(block, slot):
        return pltpu.make_async_copy(
            x_hbm.at[pl.ds(block * BLOCK_T, BLOCK_T), :],
            xbuf.at[slot],
            sems.at[slot],
        )

    for slot in range(min(NBUF, NBLOCKS)):
        copy_in(slot, slot).start()

    for block in range(NBLOCKS):
        slot = block % NBUF
        copy_in(block, slot).wait()
        xb = xbuf[slot].astype(jnp.bfloat16)
        logits = jax.lax.dot_general(
            xb, w,
            dimension_numbers=(((1,), (1,)), ((), ())),
            preferred_element_type=jnp.float32,
        )
        o_ref[pl.ds(block * BLOCK_T, BLOCK_T), :] = jnp.maximum(logits + bias, 0.0)
        nxt = block + NBUF
        if nxt < NBLOCKS:
            copy_in(nxt, slot).start()


@jax.jit
def kernel(x, W, b):
    b2 = b.reshape(1, EXPERTS)
    return pl.pallas_call(
        _router_body,
        in_specs=[
            pl.BlockSpec(memory_space=pltpu.MemorySpace.HBM),
            pl.BlockSpec(memory_space=pltpu.MemorySpace.VMEM),
            pl.BlockSpec(memory_space=pltpu.MemorySpace.VMEM),
        ],
        out_specs=pl.BlockSpec(memory_space=pltpu.MemorySpace.VMEM),
        out_shape=jax.ShapeDtypeStruct((TOKENS, EXPERTS), jnp.float32),
        scratch_shapes=[
            pltpu.VMEM((NBUF, BLOCK_T, HIDDEN), jnp.float32),
            pltpu.SemaphoreType.DMA((NBUF,)),
        ],
    )(x, W, b2)


# speedup vs baseline: 10.0893x; 1.0530x over previous
import jax
import jax.numpy as jnp
from jax.experimental import pallas as pl
from jax.experimental.pallas import tpu as pltpu

TOKENS = 16384
HIDDEN = 2048
EXPERTS = 64
BLOCK_T = 1024
KSPLIT = 2
KHALF = HIDDEN // KSPLIT


def _router_body(xa_ref, xb_ref, wa_ref, wb_ref, b_ref, o_ref):
    wa = wa_ref[...].astype(jnp.bfloat16)
    wb = wb_ref[...].astype(jnp.bfloat16)
    dn = (((1,), (1,)), ((), ()))
    acc = jax.lax.dot_general(
        xa_ref[...].astype(jnp.bfloat16), wa, dn,
        preferred_element_type=jnp.float32)
    acc = acc + jax.lax.dot_general(
        xb_ref[...].astype(jnp.bfloat16), wb, dn,
        preferred_element_type=jnp.float32)
    o_ref[...] = jnp.maximum(acc + b_ref[...], 0.0)


@jax.jit
def kernel(x, W, b):
    b2 = b.reshape(1, EXPERTS)
    grid = (TOKENS // BLOCK_T,)
    return pl.pallas_call(
        _router_body,
        grid=grid,
        in_specs=[
            pl.BlockSpec((BLOCK_T, KHALF), lambda i: (i, 0)),
            pl.BlockSpec((BLOCK_T, KHALF), lambda i: (i, 1)),
            pl.BlockSpec((EXPERTS, KHALF), lambda i: (0, 0)),
            pl.BlockSpec((EXPERTS, KHALF), lambda i: (0, 1)),
            pl.BlockSpec((1, EXPERTS), lambda i: (0, 0)),
        ],
        out_specs=pl.BlockSpec((BLOCK_T, EXPERTS), lambda i: (i, 0)),
        out_shape=jax.ShapeDtypeStruct((TOKENS, EXPERTS), jnp.float32),
        compiler_params=pltpu.CompilerParams(
            dimension_semantics=("parallel",),
        ),
    )(x, x, W, W, b2)
